# baseline (speedup 1.0000x reference)
import jax
import jax.numpy as jnp
from jax.experimental import pallas as pl
from jax.experimental.pallas import tpu as pltpu

E = 16
TOPK = 2
B = 128
D = 1024
F = 1024
EPS_PER_STEP = 2
NSTEP = E // EPS_PER_STEP


def _moe_body(x_ref, ids_ref, es_ref, w1g_ref, w1u_ref, w1sg_ref, w1su_ref,
              w2_ref, w2s_ref, out_ref, cnt_ref):
    g = pl.program_id(0)

    xv = x_ref[...].astype(jnp.bfloat16)
    acc = jnp.zeros((B, D), jnp.float32)
    for sub in range(EPS_PER_STEP):
        e = g * EPS_PER_STEP + sub
        gate = jnp.dot(xv, w1g_ref[sub].astype(jnp.bfloat16),
                       preferred_element_type=jnp.float32) * w1sg_ref[sub]
        up = jnp.dot(xv, w1u_ref[sub].astype(jnp.bfloat16),
                     preferred_element_type=jnp.float32) * w1su_ref[sub]
        h = gate * jax.nn.sigmoid(gate) * up
        y2 = jnp.dot(h.astype(jnp.bfloat16), w2_ref[sub].astype(jnp.bfloat16),
                     preferred_element_type=jnp.float32) * w2s_ref[sub]

        m = ids_ref[...] == e
        w = jnp.sum(jnp.where(m, es_ref[...], 0.0), axis=1, keepdims=True)
        acc = acc + w * y2

        cnt_ref[e] = jnp.sum(m.astype(jnp.int32))

    @pl.when(g == 0)
    def _():
        out_ref[...] = acc

    @pl.when(g != 0)
    def _():
        out_ref[...] += acc


@jax.jit
def kernel(x, expert_ids, smooth_scales, expert_scales, x_active_mask,
           gmm1_weight, gmm1_weight_scale, gmm2_weight, gmm2_weight_scale):
    del smooth_scales, x_active_mask
    w1s3 = gmm1_weight_scale.reshape(E, 1, 2 * F)
    w2s3 = gmm2_weight_scale.reshape(E, 1, D)
    P = EPS_PER_STEP

    out, counts = pl.pallas_call(
        _moe_body,
        grid=(NSTEP,),
        in_specs=[
            pl.BlockSpec((B, D), lambda g: (0, 0)),
            pl.BlockSpec((B, TOPK), lambda g: (0, 0)),
            pl.BlockSpec((B, TOPK), lambda g: (0, 0)),
            pl.BlockSpec((P, D, F), lambda g: (g, 0, 0)),
            pl.BlockSpec((P, D, F), lambda g: (g, 0, 1)),
            pl.BlockSpec((P, 1, F), lambda g: (g, 0, 0)),
            pl.BlockSpec((P, 1, F), lambda g: (g, 0, 1)),
            pl.BlockSpec((P, F, D), lambda g: (g, 0, 0)),
            pl.BlockSpec((P, 1, D), lambda g: (g, 0, 0)),
        ],
        out_specs=[
            pl.BlockSpec((B, D), lambda g: (0, 0)),
            pl.BlockSpec(memory_space=pltpu.SMEM),
        ],
        out_shape=[
            jax.ShapeDtypeStruct((B, D), jnp.float32),
            jax.ShapeDtypeStruct((E,), jnp.int32),
        ],
        compiler_params=pltpu.CompilerParams(
            dimension_semantics=("arbitrary",),
        ),
    )(x, expert_ids, expert_scales, gmm1_weight, gmm1_weight,
      w1s3, w1s3, gmm2_weight, w2s3)
    return out, counts


# speedup vs baseline: 1.0173x; 1.0173x over previous
import jax
import jax.numpy as jnp
from jax.experimental import pallas as pl
from jax.experimental.pallas import tpu as pltpu

E = 16
TOPK = 2
B = 128
D = 1024
F = 1024


def _moe_body(x_ref, ids_ref, es_ref, w1_ref, w1s_ref, w2_ref, w2s_ref,
              out_ref, cnt_ref):
    e = pl.program_id(0)

    xv = x_ref[...].astype(jnp.bfloat16)
    w1 = w1_ref[0]
    gate = jnp.dot(xv, w1[:, :F].astype(jnp.bfloat16),
                   preferred_element_type=jnp.float32) * w1s_ref[0, :, :F]
    up = jnp.dot(xv, w1[:, F:].astype(jnp.bfloat16),
                 preferred_element_type=jnp.float32) * w1s_ref[0, :, F:]
    h = gate * jax.nn.sigmoid(gate) * up
    y2 = jnp.dot(h.astype(jnp.bfloat16), w2_ref[0].astype(jnp.bfloat16),
                 preferred_element_type=jnp.float32) * w2s_ref[0]

    m = ids_ref[...] == e
    w = jnp.sum(jnp.where(m, es_ref[...], 0.0), axis=1, keepdims=True)
    cnt_ref[e] = jnp.sum(m.astype(jnp.int32))
    contrib = w * y2

    @pl.when(e == 0)
    def _():
        out_ref[...] = contrib

    @pl.when(e != 0)
    def _():
        out_ref[...] += contrib


@jax.jit
def kernel(x, expert_ids, smooth_scales, expert_scales, x_active_mask,
           gmm1_weight, gmm1_weight_scale, gmm2_weight, gmm2_weight_scale):
    del smooth_scales, x_active_mask
    w1s3 = gmm1_weight_scale.reshape(E, 1, 2 * F)
    w2s3 = gmm2_weight_scale.reshape(E, 1, D)

    out, counts = pl.pallas_call(
        _moe_body,
        grid=(E,),
        in_specs=[
            pl.BlockSpec((B, D), lambda e: (0, 0)),
            pl.BlockSpec((B, TOPK), lambda e: (0, 0)),
            pl.BlockSpec((B, TOPK), lambda e: (0, 0)),
            pl.BlockSpec((1, D, 2 * F), lambda e: (e, 0, 0)),
            pl.BlockSpec((1, 1, 2 * F), lambda e: (e, 0, 0)),
            pl.BlockSpec((1, F, D), lambda e: (e, 0, 0)),
            pl.BlockSpec((1, 1, D), lambda e: (e, 0, 0)),
        ],
        out_specs=[
            pl.BlockSpec((B, D), lambda e: (0, 0)),
            pl.BlockSpec(memory_space=pltpu.SMEM),
        ],
        out_shape=[
            jax.ShapeDtypeStruct((B, D), jnp.float32),
            jax.ShapeDtypeStruct((E,), jnp.int32),
        ],
        compiler_params=pltpu.CompilerParams(
            dimension_semantics=("arbitrary",),
        ),
    )(x, expert_ids, expert_scales, gmm1_weight, w1s3, gmm2_weight, w2s3)
    return out, counts
